# Initial kernel scaffold; baseline (speedup 1.0000x reference)
import functools

import jax
import jax.numpy as jnp
from jax import lax
from jax.experimental import pallas as pl
from jax.experimental.pallas import tpu as pltpu
from jax.experimental.pallas import tpu_sc as plsc

N = 100000
E = 3200000
NC, NS = 2, 16
NW = NC * NS
EPW = E // NW
WIN = 2000
NWIN = EPW // WIN
RPT = N // NS

_mesh = plsc.VectorSubcoreMesh(
    core_axis_name="c", subcore_axis_name="s", num_cores=NC, num_subcores=NS)


def _make_agg(C):
  @functools.partial(
      pl.kernel,
      out_type=jax.ShapeDtypeStruct((NC, N, C), jnp.float32),
      mesh=_mesh,
      scratch_types=[
          pltpu.VMEM_SHARED((N, C), jnp.float32),
          pltpu.VMEM((WIN,), jnp.int32), pltpu.VMEM((WIN,), jnp.int32),
          pltpu.VMEM((WIN,), jnp.int32), pltpu.VMEM((WIN,), jnp.int32),
          pltpu.VMEM((WIN, C), jnp.float32), pltpu.VMEM((WIN, C), jnp.float32),
          pltpu.SemaphoreType.DMA, pltpu.SemaphoreType.DMA,
          pltpu.SemaphoreType.DMA, pltpu.SemaphoreType.DMA,
          pltpu.SemaphoreType.DMA, pltpu.SemaphoreType.DMA,
      ],
  )
  def k(y_hbm, src_hbm, dst_hbm, zeros_hbm, out_hbm,
        acc, src0, src1, dst0, dst1, rows0, rows1,
        is0, is1, gs0, gs1, ss0, ss1):
    cid = lax.axis_index("c")
    sid = lax.axis_index("s")
    wid = sid * NC + cid
    ebase = wid * EPW
    srcb, dstb, rowsb = (src0, src1), (dst0, dst1), (rows0, rows1)
    isem, gsem, ssem = (is0, is1), (gs0, gs1), (ss0, ss1)

    r0 = sid * RPT
    pltpu.sync_copy(zeros_hbm.at[pl.ds(r0, RPT)], acc.at[pl.ds(r0, RPT)])
    plsc.subcore_barrier()

    def issue(w, b):
      base = ebase + w * WIN
      c1 = pltpu.async_copy(src_hbm.at[pl.ds(base, WIN)], srcb[b], isem[b])
      c2 = pltpu.async_copy(dst_hbm.at[pl.ds(base, WIN)], dstb[b], isem[b])
      c1.wait()
      c2.wait()
      pltpu.async_copy(y_hbm.at[srcb[b]], rowsb[b], gsem[b]).wait()
      pltpu.async_copy(rowsb[b], acc.at[dstb[b]], ssem[b], add=True)

    def drain(b):
      pltpu.make_async_copy(rowsb[b], acc.at[dstb[b]], ssem[b]).wait()

    for b in range(2):
      issue(b, b)

    def body(i, carry):
      for b in range(2):
        drain(b)
        issue(2 * i + b, b)
      return carry

    lax.fori_loop(1, NWIN // 2, body, 0)
    for b in range(2):
      drain(b)

    plsc.subcore_barrier()
    pltpu.sync_copy(acc.at[pl.ds(r0, RPT)], out_hbm.at[cid, pl.ds(r0, RPT)])

  return k


def _make_deg():
  ZT, ZR = 10, 10000

  @functools.partial(
      pl.kernel,
      out_type=jax.ShapeDtypeStruct((NC, N), jnp.float32),
      mesh=_mesh,
      scratch_types=[
          pltpu.VMEM_SHARED((N,), jnp.float32),
          pltpu.VMEM((WIN,), jnp.int32), pltpu.VMEM((WIN,), jnp.int32),
          pltpu.VMEM((WIN,), jnp.float32),
          pltpu.SemaphoreType.DMA, pltpu.SemaphoreType.DMA,
          pltpu.SemaphoreType.DMA, pltpu.SemaphoreType.DMA,
      ],
  )
  def k(dst_hbm, zeros_hbm, ones_hbm, out_hbm,
        acc, d0, d1, ones_v, i0, i1, s0, s1):
    cid = lax.axis_index("c")
    sid = lax.axis_index("s")
    wid = sid * NC + cid
    ebase = wid * EPW
    dstb, isem, ssem = (d0, d1), (i0, i1), (s0, s1)

    @pl.when(sid < ZT)
    def _():
      pltpu.sync_copy(zeros_hbm.at[pl.ds(sid * ZR, ZR)],
                      acc.at[pl.ds(sid * ZR, ZR)])

    pltpu.sync_copy(ones_hbm, ones_v)
    plsc.subcore_barrier()

    def issue(w, b):
      base = ebase + w * WIN
      pltpu.async_copy(dst_hbm.at[pl.ds(base, WIN)], dstb[b], isem[b]).wait()
      pltpu.async_copy(ones_v, acc.at[dstb[b]], ssem[b], add=True)

    def drain(b):
      pltpu.make_async_copy(ones_v, acc.at[dstb[b]], ssem[b]).wait()

    for b in range(2):
      issue(b, b)

    def body(i, carry):
      for b in range(2):
        drain(b)
        issue(2 * i + b, b)
      return carry

    lax.fori_loop(1, NWIN // 2, body, 0)
    for b in range(2):
      drain(b)

    plsc.subcore_barrier()

    @pl.when(sid < ZT)
    def _():
      pltpu.sync_copy(acc.at[pl.ds(sid * ZR, ZR)],
                      out_hbm.at[cid, pl.ds(sid * ZR, ZR)])

  return k


_deg = _make_deg()
_aggk = {c: _make_agg(c) for c in (4, 8, 16)}


BN = 5000


def _im(ndim, axis):
  def f(i):
    idx = [0] * ndim
    if axis is not None:
      idx[axis] = i
    return tuple(idx)
  return f


def _spec(a):
  if a.shape[0] == N:
    return pl.BlockSpec((BN,) + a.shape[1:], _im(a.ndim, 0))
  if a.ndim >= 2 and a.shape[1] == N:
    return pl.BlockSpec((a.shape[0], BN) + a.shape[2:], _im(a.ndim, 1))
  return pl.BlockSpec(a.shape, _im(a.ndim, None))


def _tc(body, out_shapes, *args):
  outs = [jax.ShapeDtypeStruct(s, jnp.float32) for s in out_shapes]
  out_specs = [pl.BlockSpec((BN,) + s[1:], _im(len(s), 0)) for s in out_shapes]
  return pl.pallas_call(
      body,
      grid=(N // BN,),
      in_specs=[_spec(a) for a in args],
      out_specs=out_specs,
      out_shape=outs,
  )(*args)


def _mm(x, w):
  return jnp.dot(x, w, preferred_element_type=jnp.float32,
                 precision=lax.Precision.HIGHEST)


def _tc0(dp, val, w1, dinv_o, y0_o):
  deg = dp[0, :] + dp[1, :] + 1.0
  dinv = lax.rsqrt(deg)[:, None]
  dinv_o[...] = dinv
  y0_o[...] = dinv * _mm(val[...], w1[...])


def _tc1(sp, y0, dinv, b1, y1_o):
  agg = dinv[...] * (sp[0] + sp[1] + y0[...])
  y1_o[...] = dinv[...] * jax.nn.relu(agg + b1[...])


def _tc2(sp, y1, dinv, w2, b2, y2_o):
  agg = dinv[...] * (sp[0] + sp[1] + y1[...])
  y2_o[...] = dinv[...] * jax.nn.relu(_mm(agg, w2[...]) + b2[...])


def _tc3(sp, y2, dinv, w3, b3, y3a_o, y3b_o):
  agg = dinv[...] * (sp[0] + sp[1] + y2[...])
  t = dinv[...] * jax.nn.relu(_mm(agg, w3[...]) + b3[...])
  y3a_o[...] = t[:, :16]
  y3b_o[...] = t[:, 16:]


def _tc4(spa, spb, y3a, y3b, dinv, w4, b4, w5, y4_o):
  agga = spa[0] + spa[1] + y3a[...]
  aggb = spb[0] + spb[1] + y3b[...]
  agg = dinv[...] * jnp.concatenate([agga, aggb], axis=1)
  h4 = jax.nn.relu(_mm(agg, w4[...]) + b4[...])
  y4_o[...] = dinv[...] * _mm(h4, w5[...])


def _tc5(sp, y4, dinv, b5, out_o):
  h5 = jax.nn.relu(dinv[...] * (sp[0] + sp[1] + y4[...]) + b5[...])
  out_o[...] = jax.nn.sigmoid(jnp.sum(h5, axis=1, keepdims=True))


def kernel(value, edge_index, W1, b1, W2, b2, W3, b3, W4, b4, W5, b5):
  src = edge_index[0]
  dst = edge_index[1]
  b1r, b2r, b3r = b1.reshape(1, -1), b2.reshape(1, -1), b3.reshape(1, -1)
  b4r, b5r = b4.reshape(1, -1), b5.reshape(1, -1)

  dp = _deg(dst, jnp.zeros((N,), jnp.float32), jnp.ones((WIN,), jnp.float32))
  dinv, y0 = _tc(_tc0, [(N, 1), (N, 8)], dp, value, W1)

  s1 = _aggk[8](y0, src, dst, jnp.zeros((N, 8), jnp.float32))
  (y1,) = _tc(_tc1, [(N, 8)], s1, y0, dinv, b1r)

  s2 = _aggk[8](y1, src, dst, jnp.zeros((N, 8), jnp.float32))
  (y2,) = _tc(_tc2, [(N, 16)], s2, y1, dinv, W2, b2r)

  s3 = _aggk[16](y2, src, dst, jnp.zeros((N, 16), jnp.float32))
  y3a, y3b = _tc(_tc3, [(N, 16), (N, 16)], s3, y2, dinv, W3, b3r)

  s4a = _aggk[16](y3a, src, dst, jnp.zeros((N, 16), jnp.float32))
  s4b = _aggk[16](y3b, src, dst, jnp.zeros((N, 16), jnp.float32))
  (y4,) = _tc(_tc4, [(N, 4)], s4a, s4b, y3a, y3b, dinv, W4, b4r, W5)

  s5 = _aggk[4](y4, src, dst, jnp.zeros((N, 4), jnp.float32))
  (out,) = _tc(_tc5, [(N, 1)], s5, y4, dinv, b5r)
  return out.reshape(-1)


# speedup vs baseline: 37.9954x; 37.9954x over previous
import functools

import jax
import jax.numpy as jnp
from jax import lax
from jax.experimental import pallas as pl
from jax.experimental.pallas import tpu as pltpu
from jax.experimental.pallas import tpu_sc as plsc

N = 100000
E = 3200000
NC, NS = 2, 16
NW = NC * NS
EPW = E // NW
WIN = 800
NWIN = EPW // WIN
RPT = N // NS

_mesh = plsc.VectorSubcoreMesh(
    core_axis_name="c", subcore_axis_name="s", num_cores=NC, num_subcores=NS)
_sc_params = pltpu.CompilerParams(use_tc_tiling_on_sc=False)


def _make_agg(C):
  @functools.partial(
      pl.kernel,
      out_type=jax.ShapeDtypeStruct((NC, N, C), jnp.float32),
      mesh=_mesh,
      compiler_params=_sc_params,
      scratch_types=[
          pltpu.VMEM_SHARED((N, C), jnp.float32),
          pltpu.VMEM((WIN,), jnp.int32), pltpu.VMEM((WIN,), jnp.int32),
          pltpu.VMEM((WIN,), jnp.int32), pltpu.VMEM((WIN,), jnp.int32),
          pltpu.VMEM((WIN, C), jnp.float32), pltpu.VMEM((WIN, C), jnp.float32),
          pltpu.SemaphoreType.DMA, pltpu.SemaphoreType.DMA,
          pltpu.SemaphoreType.DMA, pltpu.SemaphoreType.DMA,
          pltpu.SemaphoreType.DMA, pltpu.SemaphoreType.DMA,
      ],
  )
  def k(y_hbm, src_hbm, dst_hbm, zeros_hbm, out_hbm,
        acc, src0, src1, dst0, dst1, rows0, rows1,
        is0, is1, gs0, gs1, ss0, ss1):
    cid = lax.axis_index("c")
    sid = lax.axis_index("s")
    wid = sid * NC + cid
    ebase = wid * EPW
    srcb, dstb, rowsb = (src0, src1), (dst0, dst1), (rows0, rows1)
    isem, gsem, ssem = (is0, is1), (gs0, gs1), (ss0, ss1)

    r0 = sid * RPT
    pltpu.sync_copy(zeros_hbm.at[pl.ds(r0, RPT)], acc.at[pl.ds(r0, RPT)])
    plsc.subcore_barrier()

    def issue(w, b):
      base = ebase + w * WIN
      c1 = pltpu.async_copy(src_hbm.at[pl.ds(base, WIN)], srcb[b], isem[b])
      c2 = pltpu.async_copy(dst_hbm.at[pl.ds(base, WIN)], dstb[b], isem[b])
      c1.wait()
      c2.wait()
      pltpu.async_copy(y_hbm.at[srcb[b]], rowsb[b], gsem[b]).wait()
      pltpu.async_copy(rowsb[b], acc.at[dstb[b]], ssem[b], add=True)

    def drain(b):
      pltpu.make_async_copy(rowsb[b], acc.at[dstb[b]], ssem[b]).wait()

    for b in range(2):
      issue(b, b)

    def body(i, carry):
      for b in range(2):
        drain(b)
        issue(2 * i + b, b)
      return carry

    lax.fori_loop(1, NWIN // 2, body, 0)
    for w in range(2 * (NWIN // 2), NWIN):
      drain(w % 2)
      issue(w, w % 2)
    for b in range(2):
      drain(b)

    plsc.subcore_barrier()
    pltpu.sync_copy(acc.at[pl.ds(r0, RPT)], out_hbm.at[cid, pl.ds(r0, RPT)])

  return k


def _make_deg():
  ZT, ZR = 10, 10000

  @functools.partial(
      pl.kernel,
      out_type=jax.ShapeDtypeStruct((NC, N, 1), jnp.float32),
      mesh=_mesh,
      compiler_params=_sc_params,
      scratch_types=[
          pltpu.VMEM_SHARED((N, 1), jnp.float32),
          pltpu.VMEM((WIN,), jnp.int32), pltpu.VMEM((WIN,), jnp.int32),
          pltpu.VMEM((WIN, 1), jnp.float32),
          pltpu.SemaphoreType.DMA, pltpu.SemaphoreType.DMA,
          pltpu.SemaphoreType.DMA, pltpu.SemaphoreType.DMA,
      ],
  )
  def k(dst_hbm, zeros_hbm, ones_hbm, out_hbm,
        acc, d0, d1, ones_v, i0, i1, s0, s1):
    cid = lax.axis_index("c")
    sid = lax.axis_index("s")
    wid = sid * NC + cid
    ebase = wid * EPW
    dstb, isem, ssem = (d0, d1), (i0, i1), (s0, s1)

    @pl.when(sid < ZT)
    def _():
      pltpu.sync_copy(zeros_hbm.at[pl.ds(sid * ZR, ZR)],
                      acc.at[pl.ds(sid * ZR, ZR)])

    pltpu.sync_copy(ones_hbm, ones_v)
    plsc.subcore_barrier()

    def issue(w, b):
      base = ebase + w * WIN
      pltpu.async_copy(dst_hbm.at[pl.ds(base, WIN)], dstb[b], isem[b]).wait()
      pltpu.async_copy(ones_v, acc.at[dstb[b]], ssem[b], add=True)

    def drain(b):
      pltpu.make_async_copy(ones_v, acc.at[dstb[b]], ssem[b]).wait()

    for b in range(2):
      issue(b, b)

    def body(i, carry):
      for b in range(2):
        drain(b)
        issue(2 * i + b, b)
      return carry

    lax.fori_loop(1, NWIN // 2, body, 0)
    for w in range(2 * (NWIN // 2), NWIN):
      drain(w % 2)
      issue(w, w % 2)
    for b in range(2):
      drain(b)

    plsc.subcore_barrier()

    @pl.when(sid < ZT)
    def _():
      pltpu.sync_copy(acc.at[pl.ds(sid * ZR, ZR)],
                      out_hbm.at[cid, pl.ds(sid * ZR, ZR)])

  return k


_deg = _make_deg()
_aggk = {c: _make_agg(c) for c in (4, 8, 16)}


BN = 5000


def _im(ndim, axis):
  def f(i):
    idx = [0] * ndim
    if axis is not None:
      idx[axis] = i
    return tuple(idx)
  return f


def _spec(a):
  if a.shape[0] == N:
    return pl.BlockSpec((BN,) + a.shape[1:], _im(a.ndim, 0))
  if a.ndim >= 2 and a.shape[1] == N:
    return pl.BlockSpec((a.shape[0], BN) + a.shape[2:], _im(a.ndim, 1))
  return pl.BlockSpec(a.shape, _im(a.ndim, None))


def _tc(body, out_shapes, *args):
  outs = [jax.ShapeDtypeStruct(s, jnp.float32) for s in out_shapes]
  out_specs = [pl.BlockSpec((BN,) + s[1:], _im(len(s), 0)) for s in out_shapes]
  return pl.pallas_call(
      body,
      grid=(N // BN,),
      in_specs=[_spec(a) for a in args],
      out_specs=out_specs,
      out_shape=outs,
  )(*args)


def _mm(x, w):
  return jnp.dot(x, w, preferred_element_type=jnp.float32,
                 precision=lax.Precision.HIGHEST)


def _tc0(dp, val, w1, dinv_o, y0_o):
  deg = dp[0] + dp[1] + 1.0
  dinv = lax.rsqrt(deg)
  dinv_o[...] = dinv
  y0_o[...] = dinv * _mm(val[...], w1[...])


def _tc1(sp, y0, dinv, b1, y1_o):
  agg = dinv[...] * (sp[0] + sp[1] + y0[...])
  y1_o[...] = dinv[...] * jax.nn.relu(agg + b1[...])


def _tc2(sp, y1, dinv, w2, b2, y2_o):
  agg = dinv[...] * (sp[0] + sp[1] + y1[...])
  y2_o[...] = dinv[...] * jax.nn.relu(_mm(agg, w2[...]) + b2[...])


def _tc3(sp, y2, dinv, w3, b3, y3a_o, y3b_o):
  agg = dinv[...] * (sp[0] + sp[1] + y2[...])
  t = dinv[...] * jax.nn.relu(_mm(agg, w3[...]) + b3[...])
  y3a_o[...] = t[:, :16]
  y3b_o[...] = t[:, 16:]


def _tc4(spa, spb, y3a, y3b, dinv, w4, b4, w5, y4_o):
  agga = spa[0] + spa[1] + y3a[...]
  aggb = spb[0] + spb[1] + y3b[...]
  agg = dinv[...] * jnp.concatenate([agga, aggb], axis=1)
  h4 = jax.nn.relu(_mm(agg, w4[...]) + b4[...])
  y4_o[...] = dinv[...] * _mm(h4, w5[...])


def _tc5(sp, y4, dinv, b5, out_o):
  h5 = jax.nn.relu(dinv[...] * (sp[0] + sp[1] + y4[...]) + b5[...])
  out_o[...] = jax.nn.sigmoid(jnp.sum(h5, axis=1, keepdims=True))


def kernel(value, edge_index, W1, b1, W2, b2, W3, b3, W4, b4, W5, b5):
  src = edge_index[0]
  dst = edge_index[1]
  b1r, b2r, b3r = b1.reshape(1, -1), b2.reshape(1, -1), b3.reshape(1, -1)
  b4r, b5r = b4.reshape(1, -1), b5.reshape(1, -1)

  dp = _deg(dst, jnp.zeros((N, 1), jnp.float32),
            jnp.ones((WIN, 1), jnp.float32))
  dinv, y0 = _tc(_tc0, [(N, 1), (N, 8)], dp, value, W1)

  s1 = _aggk[8](y0, src, dst, jnp.zeros((N, 8), jnp.float32))
  (y1,) = _tc(_tc1, [(N, 8)], s1, y0, dinv, b1r)

  s2 = _aggk[8](y1, src, dst, jnp.zeros((N, 8), jnp.float32))
  (y2,) = _tc(_tc2, [(N, 16)], s2, y1, dinv, W2, b2r)

  s3 = _aggk[16](y2, src, dst, jnp.zeros((N, 16), jnp.float32))
  y3a, y3b = _tc(_tc3, [(N, 16), (N, 16)], s3, y2, dinv, W3, b3r)

  s4a = _aggk[16](y3a, src, dst, jnp.zeros((N, 16), jnp.float32))
  s4b = _aggk[16](y3b, src, dst, jnp.zeros((N, 16), jnp.float32))
  (y4,) = _tc(_tc4, [(N, 4)], s4a, s4b, y3a, y3b, dinv, W4, b4r, W5)

  s5 = _aggk[4](y4, src, dst, jnp.zeros((N, 4), jnp.float32))
  (out,) = _tc(_tc5, [(N, 1)], s5, y4, dinv, b5r)
  return out.reshape(-1)


# speedup vs baseline: 50.9060x; 1.3398x over previous
import functools

import jax
import jax.numpy as jnp
from jax import lax
from jax.experimental import pallas as pl
from jax.experimental.pallas import tpu as pltpu
from jax.experimental.pallas import tpu_sc as plsc

N = 100000
E = 3200000
NC, NS = 2, 16
NW = NC * NS
EPW = E // NW
WD = 5000
NWD = EPW // WD
RPT = N // NS

_mesh = plsc.VectorSubcoreMesh(
    core_axis_name="c", subcore_axis_name="s", num_cores=NC, num_subcores=NS)
_sc_params = pltpu.CompilerParams(use_tc_tiling_on_sc=False)


def _make_agg(C, WINC):
  NWINC = EPW // WINC
  NITER = (NWINC + 5) // 6

  @functools.partial(
      pl.kernel,
      out_type=jax.ShapeDtypeStruct((NC, N, C), jnp.float32),
      mesh=_mesh,
      compiler_params=_sc_params,
      scratch_types=[
          pltpu.VMEM_SHARED((N, C), jnp.float32),
          pltpu.VMEM((WINC,), jnp.int32), pltpu.VMEM((WINC,), jnp.int32),
          pltpu.VMEM((WINC,), jnp.int32), pltpu.VMEM((WINC,), jnp.int32),
          pltpu.VMEM((WINC,), jnp.int32),
          pltpu.VMEM((WINC, C), jnp.float32),
          pltpu.VMEM((WINC, C), jnp.float32),
          pltpu.SemaphoreType.DMA, pltpu.SemaphoreType.DMA,
          pltpu.SemaphoreType.DMA, pltpu.SemaphoreType.DMA,
          pltpu.SemaphoreType.DMA,
          pltpu.SemaphoreType.DMA, pltpu.SemaphoreType.DMA,
          pltpu.SemaphoreType.DMA, pltpu.SemaphoreType.DMA,
          pltpu.SemaphoreType.DMA,
      ],
  )
  def k(y_hbm, src_hbm, dst_hbm, zeros_hbm, out_hbm,
        acc, src0, src1, dst0, dst1, dst2, rows0, rows1,
        es0, es1, ed0, ed1, ed2, gs0, gs1, ss0, ss1, ss2):
    cid = lax.axis_index("c")
    sid = lax.axis_index("s")
    wid = sid * NC + cid
    ebase = wid * EPW
    srcb, dstb, rowsb = (src0, src1), (dst0, dst1, dst2), (rows0, rows1)
    ssem_, dsem = (es0, es1), (ed0, ed1, ed2)
    gsem, ssem = (gs0, gs1), (ss0, ss1, ss2)

    r0 = sid * RPT
    pltpu.sync_copy(zeros_hbm.at[pl.ds(r0, RPT)], acc.at[pl.ds(r0, RPT)])
    plsc.subcore_barrier()

    def copy_src(w, b):
      pltpu.async_copy(src_hbm.at[pl.ds(ebase + w * WINC, WINC)],
                       srcb[b], ssem_[b])

    def copy_dst(w, t):
      pltpu.async_copy(dst_hbm.at[pl.ds(ebase + w * WINC, WINC)],
                       dstb[t], dsem[t])

    def wait_src(b):
      pltpu.make_async_copy(src_hbm.at[pl.ds(ebase, WINC)],
                            srcb[b], ssem_[b]).wait()

    def wait_dst(t):
      pltpu.make_async_copy(dst_hbm.at[pl.ds(ebase, WINC)],
                            dstb[t], dsem[t]).wait()

    def gather(b):
      pltpu.async_copy(y_hbm.at[srcb[b]], rowsb[b], gsem[b])

    def wait_gather(b):
      pltpu.make_async_copy(y_hbm.at[srcb[b]], rowsb[b], gsem[b]).wait()

    def scat(b, t):
      pltpu.async_copy(rowsb[b], acc.at[dstb[t]], ssem[t], add=True)

    def wait_scat(b, t):
      pltpu.make_async_copy(rowsb[b], acc.at[dstb[t]], ssem[t]).wait()

    copy_src(0, 0)
    copy_dst(0, 0)
    copy_src(1, 1)
    copy_dst(1, 1)
    wait_src(0)
    gather(0)

    def step(w, b, t):
      nb, nt2 = 1 - b, (t + 2) % 3

      @pl.when(w + 1 < NWINC)
      def _():
        wait_src(nb)

        @pl.when(w >= 1)
        def _():
          wait_scat(nb, nt2)

        gather(nb)

      @pl.when(w < NWINC)
      def _():
        wait_gather(b)
        wait_dst(t)
        scat(b, t)

      @pl.when(w + 2 < NWINC)
      def _():
        copy_src(w + 2, b)
        copy_dst(w + 2, nt2)

    def body(i, carry):
      for j in range(6):
        step(6 * i + j, j % 2, j % 3)
      return carry

    lax.fori_loop(0, NITER, body, 0)
    wait_scat((NWINC - 1) % 2, (NWINC - 1) % 3)

    plsc.subcore_barrier()
    pltpu.sync_copy(acc.at[pl.ds(r0, RPT)], out_hbm.at[cid, pl.ds(r0, RPT)])

  return k


def _make_deg():
  ZT, ZR = 10, 10000

  @functools.partial(
      pl.kernel,
      out_type=jax.ShapeDtypeStruct((NC, N, 1), jnp.float32),
      mesh=_mesh,
      compiler_params=_sc_params,
      scratch_types=[
          pltpu.VMEM_SHARED((N, 1), jnp.float32),
          pltpu.VMEM((WD,), jnp.int32), pltpu.VMEM((WD,), jnp.int32),
          pltpu.VMEM((WD, 1), jnp.float32),
          pltpu.SemaphoreType.DMA, pltpu.SemaphoreType.DMA,
          pltpu.SemaphoreType.DMA, pltpu.SemaphoreType.DMA,
      ],
  )
  def k(dst_hbm, zeros_hbm, ones_hbm, out_hbm,
        acc, d0, d1, ones_v, i0, i1, s0, s1):
    cid = lax.axis_index("c")
    sid = lax.axis_index("s")
    wid = sid * NC + cid
    ebase = wid * EPW
    dstb, isem, ssem = (d0, d1), (i0, i1), (s0, s1)

    @pl.when(sid < ZT)
    def _():
      pltpu.sync_copy(zeros_hbm.at[pl.ds(sid * ZR, ZR)],
                      acc.at[pl.ds(sid * ZR, ZR)])

    pltpu.sync_copy(ones_hbm, ones_v)
    plsc.subcore_barrier()

    def issue(w, b):
      base = ebase + w * WD
      pltpu.async_copy(dst_hbm.at[pl.ds(base, WD)], dstb[b], isem[b]).wait()
      pltpu.async_copy(ones_v, acc.at[dstb[b]], ssem[b], add=True)

    def drain(b):
      pltpu.make_async_copy(ones_v, acc.at[dstb[b]], ssem[b]).wait()

    for b in range(2):
      issue(b, b)

    def body(i, carry):
      for b in range(2):
        drain(b)
        issue(2 * i + b, b)
      return carry

    lax.fori_loop(1, NWD // 2, body, 0)
    for b in range(2):
      drain(b)

    plsc.subcore_barrier()

    @pl.when(sid < ZT)
    def _():
      pltpu.sync_copy(acc.at[pl.ds(sid * ZR, ZR)],
                      out_hbm.at[cid, pl.ds(sid * ZR, ZR)])

  return k


_deg = _make_deg()
_aggk = {4: _make_agg(4, 5000), 8: _make_agg(8, 2000), 16: _make_agg(16, 800)}


BN = 5000


def _im(ndim, axis):
  def f(i):
    idx = [0] * ndim
    if axis is not None:
      idx[axis] = i
    return tuple(idx)
  return f


def _spec(a):
  if a.shape[0] == N:
    return pl.BlockSpec((BN,) + a.shape[1:], _im(a.ndim, 0))
  if a.ndim >= 2 and a.shape[1] == N:
    return pl.BlockSpec((a.shape[0], BN) + a.shape[2:], _im(a.ndim, 1))
  return pl.BlockSpec(a.shape, _im(a.ndim, None))


def _tc(body, out_shapes, *args):
  outs = [jax.ShapeDtypeStruct(s, jnp.float32) for s in out_shapes]
  out_specs = [pl.BlockSpec((BN,) + s[1:], _im(len(s), 0)) for s in out_shapes]
  return pl.pallas_call(
      body,
      grid=(N // BN,),
      in_specs=[_spec(a) for a in args],
      out_specs=out_specs,
      out_shape=outs,
  )(*args)


def _mm(x, w):
  return jnp.dot(x, w, preferred_element_type=jnp.float32,
                 precision=lax.Precision.HIGHEST)


def _tc0(dp, val, w1, dinv_o, y0_o):
  deg = dp[0] + dp[1] + 1.0
  dinv = lax.rsqrt(deg)
  dinv_o[...] = dinv
  y0_o[...] = dinv * _mm(val[...], w1[...])


def _tc1(sp, y0, dinv, b1, y1_o):
  agg = dinv[...] * (sp[0] + sp[1] + y0[...])
  y1_o[...] = dinv[...] * jax.nn.relu(agg + b1[...])


def _tc2(sp, y1, dinv, w2, b2, y2_o):
  agg = dinv[...] * (sp[0] + sp[1] + y1[...])
  y2_o[...] = dinv[...] * jax.nn.relu(_mm(agg, w2[...]) + b2[...])


def _tc3(sp, y2, dinv, w3, b3, y3a_o, y3b_o):
  agg = dinv[...] * (sp[0] + sp[1] + y2[...])
  t = dinv[...] * jax.nn.relu(_mm(agg, w3[...]) + b3[...])
  y3a_o[...] = t[:, :16]
  y3b_o[...] = t[:, 16:]


def _tc4(spa, spb, y3a, y3b, dinv, w4, b4, w5, y4_o):
  agga = spa[0] + spa[1] + y3a[...]
  aggb = spb[0] + spb[1] + y3b[...]
  agg = dinv[...] * jnp.concatenate([agga, aggb], axis=1)
  h4 = jax.nn.relu(_mm(agg, w4[...]) + b4[...])
  y4_o[...] = dinv[...] * _mm(h4, w5[...])


def _tc5(sp, y4, dinv, b5, out_o):
  h5 = jax.nn.relu(dinv[...] * (sp[0] + sp[1] + y4[...]) + b5[...])
  out_o[...] = jax.nn.sigmoid(jnp.sum(h5, axis=1, keepdims=True))


def kernel(value, edge_index, W1, b1, W2, b2, W3, b3, W4, b4, W5, b5):
  src = edge_index[0]
  dst = edge_index[1]
  b1r, b2r, b3r = b1.reshape(1, -1), b2.reshape(1, -1), b3.reshape(1, -1)
  b4r, b5r = b4.reshape(1, -1), b5.reshape(1, -1)

  dp = _deg(dst, jnp.zeros((N, 1), jnp.float32),
            jnp.ones((WD, 1), jnp.float32))
  dinv, y0 = _tc(_tc0, [(N, 1), (N, 8)], dp, value, W1)

  s1 = _aggk[8](y0, src, dst, jnp.zeros((N, 8), jnp.float32))
  (y1,) = _tc(_tc1, [(N, 8)], s1, y0, dinv, b1r)

  s2 = _aggk[8](y1, src, dst, jnp.zeros((N, 8), jnp.float32))
  (y2,) = _tc(_tc2, [(N, 16)], s2, y1, dinv, W2, b2r)

  s3 = _aggk[16](y2, src, dst, jnp.zeros((N, 16), jnp.float32))
  y3a, y3b = _tc(_tc3, [(N, 16), (N, 16)], s3, y2, dinv, W3, b3r)

  s4a = _aggk[16](y3a, src, dst, jnp.zeros((N, 16), jnp.float32))
  s4b = _aggk[16](y3b, src, dst, jnp.zeros((N, 16), jnp.float32))
  (y4,) = _tc(_tc4, [(N, 4)], s4a, s4b, y3a, y3b, dinv, W4, b4r, W5)

  s5 = _aggk[4](y4, src, dst, jnp.zeros((N, 4), jnp.float32))
  (out,) = _tc(_tc5, [(N, 1)], s5, y4, dinv, b5r)
  return out.reshape(-1)


# speedup vs baseline: 53.1161x; 1.0434x over previous
import functools

import jax
import jax.numpy as jnp
from jax import lax
from jax.experimental import pallas as pl
from jax.experimental.pallas import tpu as pltpu
from jax.experimental.pallas import tpu_sc as plsc

N = 100000
E = 3200000
NC, NS = 2, 16
NW = NC * NS
EPW = E // NW
WD = 5000
NWD = EPW // WD
RPT = N // NS

_mesh = plsc.VectorSubcoreMesh(
    core_axis_name="c", subcore_axis_name="s", num_cores=NC, num_subcores=NS)
_sc_params = pltpu.CompilerParams(use_tc_tiling_on_sc=False)


def _make_agg(C, WINC):
  NWINC = EPW // WINC
  NITER = (NWINC + 5) // 6

  @functools.partial(
      pl.kernel,
      out_type=jax.ShapeDtypeStruct((NC, N, C), jnp.float32),
      mesh=_mesh,
      compiler_params=_sc_params,
      scratch_types=[
          pltpu.VMEM_SHARED((N, C), jnp.float32),
          pltpu.VMEM((WINC,), jnp.int32), pltpu.VMEM((WINC,), jnp.int32),
          pltpu.VMEM((WINC,), jnp.int32), pltpu.VMEM((WINC,), jnp.int32),
          pltpu.VMEM((WINC,), jnp.int32),
          pltpu.VMEM((WINC, C), jnp.float32),
          pltpu.VMEM((WINC, C), jnp.float32),
          pltpu.SemaphoreType.DMA, pltpu.SemaphoreType.DMA,
          pltpu.SemaphoreType.DMA, pltpu.SemaphoreType.DMA,
          pltpu.SemaphoreType.DMA,
          pltpu.SemaphoreType.DMA, pltpu.SemaphoreType.DMA,
          pltpu.SemaphoreType.DMA, pltpu.SemaphoreType.DMA,
          pltpu.SemaphoreType.DMA,
      ],
  )
  def k(y_hbm, src_hbm, dst_hbm, zeros_hbm, out_hbm,
        acc, src0, src1, dst0, dst1, dst2, rows0, rows1,
        es0, es1, ed0, ed1, ed2, gs0, gs1, ss0, ss1, ss2):
    cid = lax.axis_index("c")
    sid = lax.axis_index("s")
    wid = sid * NC + cid
    ebase = wid * EPW
    srcb, dstb, rowsb = (src0, src1), (dst0, dst1, dst2), (rows0, rows1)
    ssem_, dsem = (es0, es1), (ed0, ed1, ed2)
    gsem, ssem = (gs0, gs1), (ss0, ss1, ss2)

    r0 = sid * RPT
    pltpu.sync_copy(zeros_hbm.at[pl.ds(r0, RPT)], acc.at[pl.ds(r0, RPT)])
    plsc.subcore_barrier()

    def copy_src(w, b):
      pltpu.async_copy(src_hbm.at[pl.ds(ebase + w * WINC, WINC)],
                       srcb[b], ssem_[b])

    def copy_dst(w, t):
      pltpu.async_copy(dst_hbm.at[pl.ds(ebase + w * WINC, WINC)],
                       dstb[t], dsem[t])

    def wait_src(b):
      pltpu.make_async_copy(src_hbm.at[pl.ds(ebase, WINC)],
                            srcb[b], ssem_[b]).wait()

    def wait_dst(t):
      pltpu.make_async_copy(dst_hbm.at[pl.ds(ebase, WINC)],
                            dstb[t], dsem[t]).wait()

    def gather(b):
      pltpu.async_copy(y_hbm.at[srcb[b]], rowsb[b], gsem[b])

    def wait_gather(b):
      pltpu.make_async_copy(y_hbm.at[srcb[b]], rowsb[b], gsem[b]).wait()

    def scat(b, t):
      pltpu.async_copy(rowsb[b], acc.at[dstb[t]], ssem[t], add=True)

    def wait_scat(b, t):
      pltpu.make_async_copy(rowsb[b], acc.at[dstb[t]], ssem[t]).wait()

    copy_src(0, 0)
    copy_dst(0, 0)
    copy_src(1, 1)
    copy_dst(1, 1)
    wait_src(0)
    gather(0)

    def step(w, b, t):
      nb, nt2 = 1 - b, (t + 2) % 3

      @pl.when(w + 1 < NWINC)
      def _():
        wait_src(nb)

        @pl.when(w >= 1)
        def _():
          wait_scat(nb, nt2)

        gather(nb)

      @pl.when(w < NWINC)
      def _():
        wait_gather(b)
        wait_dst(t)
        scat(b, t)

      @pl.when(w + 2 < NWINC)
      def _():
        copy_src(w + 2, b)
        copy_dst(w + 2, nt2)

    def body(i, carry):
      for j in range(6):
        step(6 * i + j, j % 2, j % 3)
      return carry

    lax.fori_loop(0, NITER, body, 0)
    wait_scat((NWINC - 2) % 2, (NWINC - 2) % 3)
    wait_scat((NWINC - 1) % 2, (NWINC - 1) % 3)

    plsc.subcore_barrier()
    pltpu.sync_copy(acc.at[pl.ds(r0, RPT)], out_hbm.at[cid, pl.ds(r0, RPT)])

  return k


def _make_deg():
  ZT, ZR = 10, 10000

  @functools.partial(
      pl.kernel,
      out_type=jax.ShapeDtypeStruct((NC, N, 8), jnp.float32),
      mesh=_mesh,
      compiler_params=_sc_params,
      scratch_types=[
          pltpu.VMEM_SHARED((N, 8), jnp.float32),
          pltpu.VMEM((WD,), jnp.int32), pltpu.VMEM((WD,), jnp.int32),
          pltpu.VMEM((WD, 8), jnp.float32),
          pltpu.SemaphoreType.DMA, pltpu.SemaphoreType.DMA,
          pltpu.SemaphoreType.DMA, pltpu.SemaphoreType.DMA,
      ],
  )
  def k(dst_hbm, zeros_hbm, ones_hbm, out_hbm,
        acc, d0, d1, ones_v, i0, i1, s0, s1):
    cid = lax.axis_index("c")
    sid = lax.axis_index("s")
    wid = sid * NC + cid
    ebase = wid * EPW
    dstb, isem, ssem = (d0, d1), (i0, i1), (s0, s1)

    @pl.when(sid < ZT)
    def _():
      pltpu.sync_copy(zeros_hbm.at[pl.ds(sid * ZR, ZR)],
                      acc.at[pl.ds(sid * ZR, ZR)])

    pltpu.sync_copy(ones_hbm, ones_v)
    plsc.subcore_barrier()

    def issue(w, b):
      base = ebase + w * WD
      pltpu.async_copy(dst_hbm.at[pl.ds(base, WD)], dstb[b], isem[b]).wait()
      pltpu.async_copy(ones_v, acc.at[dstb[b]], ssem[b], add=True)

    def drain(b):
      pltpu.make_async_copy(ones_v, acc.at[dstb[b]], ssem[b]).wait()

    for b in range(2):
      issue(b, b)

    def body(i, carry):
      for b in range(2):
        drain(b)
        issue(2 * i + b, b)
      return carry

    lax.fori_loop(1, NWD // 2, body, 0)
    for b in range(2):
      drain(b)

    plsc.subcore_barrier()

    @pl.when(sid < ZT)
    def _():
      pltpu.sync_copy(acc.at[pl.ds(sid * ZR, ZR)],
                      out_hbm.at[cid, pl.ds(sid * ZR, ZR)])

  return k


_deg = _make_deg()
_aggk = {8: _make_agg(8, 2000), 16: _make_agg(16, 800)}


BN = 5000


def _im(ndim, axis):
  def f(i):
    idx = [0] * ndim
    if axis is not None:
      idx[axis] = i
    return tuple(idx)
  return f


def _spec(a):
  if a.shape[0] == N:
    return pl.BlockSpec((BN,) + a.shape[1:], _im(a.ndim, 0))
  if a.ndim >= 2 and a.shape[1] == N:
    return pl.BlockSpec((a.shape[0], BN) + a.shape[2:], _im(a.ndim, 1))
  return pl.BlockSpec(a.shape, _im(a.ndim, None))


def _tc(body, out_shapes, *args):
  outs = [jax.ShapeDtypeStruct(s, jnp.float32) for s in out_shapes]
  out_specs = [pl.BlockSpec((BN,) + s[1:], _im(len(s), 0)) for s in out_shapes]
  return pl.pallas_call(
      body,
      grid=(N // BN,),
      in_specs=[_spec(a) for a in args],
      out_specs=out_specs,
      out_shape=outs,
  )(*args)


def _mm(x, w):
  return jnp.dot(x, w, preferred_element_type=jnp.float32,
                 precision=lax.Precision.HIGHEST)


def _tc0(dp, val, w1, dinv_o, y0_o):
  deg = dp[0][:, :1] + dp[1][:, :1] + 1.0
  dinv = lax.rsqrt(deg)
  dinv_o[...] = dinv
  y0_o[...] = dinv * _mm(val[...], w1[...])


def _tc1(sp, y0, dinv, b1, y1_o):
  agg = dinv[...] * (sp[0] + sp[1] + y0[...])
  y1_o[...] = dinv[...] * jax.nn.relu(agg + b1[...])


def _tc2(sp, y1, dinv, w2, b2, y2_o):
  agg = dinv[...] * (sp[0] + sp[1] + y1[...])
  y2_o[...] = dinv[...] * jax.nn.relu(_mm(agg, w2[...]) + b2[...])


def _tc3(sp, y2, dinv, w3, b3, y3a_o, y3b_o):
  agg = dinv[...] * (sp[0] + sp[1] + y2[...])
  t = dinv[...] * jax.nn.relu(_mm(agg, w3[...]) + b3[...])
  y3a_o[...] = t[:, :16]
  y3b_o[...] = t[:, 16:]


def _tc4(spa, spb, y3a, y3b, dinv, w4, b4, w5, y4_o):
  agga = spa[0] + spa[1] + y3a[...]
  aggb = spb[0] + spb[1] + y3b[...]
  agg = dinv[...] * jnp.concatenate([agga, aggb], axis=1)
  h4 = jax.nn.relu(_mm(agg, w4[...]) + b4[...])
  t = dinv[...] * _mm(h4, w5[...])
  y4_o[...] = jnp.concatenate([t, jnp.zeros_like(t)], axis=1)


def _tc5(sp, y4, dinv, b5, out_o):
  s = (sp[0] + sp[1] + y4[...])[:, :4]
  h5 = jax.nn.relu(dinv[...] * s + b5[...])
  out_o[...] = jax.nn.sigmoid(jnp.sum(h5, axis=1, keepdims=True))


def kernel(value, edge_index, W1, b1, W2, b2, W3, b3, W4, b4, W5, b5):
  src = edge_index[0]
  dst = edge_index[1]
  b1r, b2r, b3r = b1.reshape(1, -1), b2.reshape(1, -1), b3.reshape(1, -1)
  b4r, b5r = b4.reshape(1, -1), b5.reshape(1, -1)

  dp = _deg(dst, jnp.zeros((N, 8), jnp.float32),
            jnp.ones((WD, 8), jnp.float32))
  dinv, y0 = _tc(_tc0, [(N, 1), (N, 8)], dp, value, W1)

  s1 = _aggk[8](y0, src, dst, jnp.zeros((N, 8), jnp.float32))
  (y1,) = _tc(_tc1, [(N, 8)], s1, y0, dinv, b1r)

  s2 = _aggk[8](y1, src, dst, jnp.zeros((N, 8), jnp.float32))
  (y2,) = _tc(_tc2, [(N, 16)], s2, y1, dinv, W2, b2r)

  s3 = _aggk[16](y2, src, dst, jnp.zeros((N, 16), jnp.float32))
  y3a, y3b = _tc(_tc3, [(N, 16), (N, 16)], s3, y2, dinv, W3, b3r)

  s4a = _aggk[16](y3a, src, dst, jnp.zeros((N, 16), jnp.float32))
  s4b = _aggk[16](y3b, src, dst, jnp.zeros((N, 16), jnp.float32))
  (y4,) = _tc(_tc4, [(N, 8)], s4a, s4b, y3a, y3b, dinv, W4, b4r, W5)

  s5 = _aggk[8](y4, src, dst, jnp.zeros((N, 8), jnp.float32))
  (out,) = _tc(_tc5, [(N, 1)], s5, y4, dinv, b5r)
  return out.reshape(-1)
